# baseline (speedup 1.0000x reference)
import functools

import jax
import jax.numpy as jnp
from jax import lax
from jax.experimental import pallas as pl
from jax.experimental.pallas import tpu as pltpu
from jax.experimental.pallas import tpu_sc as plsc

BATCH = 4096
SEQ = 200
EMB = 64
NCLS = 8
VOCAB = 1000000

NC = 2
NS = 16
NW = NC * NS
ROWS_PER_W = BATCH // NW
CHUNK = SEQ // 2
L = 16
EV = EMB // L

NBUF = 2

_mesh = plsc.VectorSubcoreMesh(
    core_axis_name="c", subcore_axis_name="s", num_cores=NC, num_subcores=NS)


@functools.partial(
    pl.kernel,
    out_type=jax.ShapeDtypeStruct((BATCH, EMB), jnp.float32),
    mesh=_mesh,
    scratch_types=[
        pltpu.VMEM((ROWS_PER_W * 2, CHUNK), jnp.int32),
        pltpu.VMEM((NBUF, 2, CHUNK, EMB), jnp.float32),
        pltpu.VMEM((ROWS_PER_W, EMB), jnp.float32),
        [pltpu.SemaphoreType.DMA] * NBUF,
    ],
    compiler_params=pltpu.CompilerParams(use_tc_tiling_on_sc=False),
)
def _pool_kernel(idx_hbm, table_hbm, out_hbm, idx_v, buf, out_v, sems):
    wid = lax.axis_index("s") * NC + lax.axis_index("c")
    base2 = wid * (ROWS_PER_W * 2)
    pltpu.sync_copy(idx_hbm.at[pl.ds(base2, ROWS_PER_W * 2)], idx_v)

    neg = jnp.full((L,), -jnp.inf, dtype=jnp.float32)

    def fire(row, slot):
        pltpu.async_copy(table_hbm.at[idx_v.at[2 * row]], buf.at[slot, 0],
                         sems[slot])
        pltpu.async_copy(table_hbm.at[idx_v.at[2 * row + 1]], buf.at[slot, 1],
                         sems[slot])

    def drain(slot):
        for c in range(2):
            pltpu.make_async_copy(table_hbm.at[pl.ds(0, CHUNK)],
                                  buf.at[slot, c], sems[slot]).wait()

    def chunk_max(bufc, acc):
        def jbody(j, a):
            return tuple(
                jnp.maximum(a[d], bufc[j, pl.ds(L * d, L)])
                for d in range(EV))
        return lax.fori_loop(0, CHUNK, jbody, acc)

    for slot in range(NBUF):
        fire(slot, slot)

    def outer_body(k, carry):
        for p in range(NBUF):
            r = NBUF * k + p
            drain(p)
            acc = chunk_max(buf.at[p, 0], (neg,) * EV)
            acc = chunk_max(buf.at[p, 1], acc)
            for d in range(EV):
                out_v[r, pl.ds(L * d, L)] = acc[d]
            fire(lax.rem(r + NBUF, ROWS_PER_W), p)
        return carry

    lax.fori_loop(0, ROWS_PER_W // NBUF, outer_body, 0)
    for slot in range(NBUF):
        drain(slot)
    pltpu.sync_copy(out_v, out_hbm.at[pl.ds(wid * ROWS_PER_W, ROWS_PER_W)])


def _matmul_body(x_ref, wt_ref, b_ref, o_ref):
    o_ref[:] = (
        jnp.dot(x_ref[:], wt_ref[:], preferred_element_type=jnp.float32)
        + b_ref[:])


_matmul = pl.pallas_call(
    _matmul_body,
    out_shape=jax.ShapeDtypeStruct((BATCH, NCLS), jnp.float32),
)


def kernel(content, emb_table, W, b):
    idx = content.reshape(BATCH * 2, CHUNK)
    pooled = _pool_kernel(idx, emb_table)
    return _matmul(pooled, W.T, b.reshape(1, NCLS))


# speedup vs baseline: 1.0722x; 1.0722x over previous
import functools

import jax
import jax.numpy as jnp
from jax import lax
from jax.experimental import pallas as pl
from jax.experimental.pallas import tpu as pltpu
from jax.experimental.pallas import tpu_sc as plsc

BATCH = 4096
SEQ = 200
EMB = 64
NCLS = 8
VOCAB = 1000000

NC = 2
NS = 16
NW = NC * NS
ROWS_PER_W = BATCH // NW
CHUNK = SEQ // 2
L = 16
EV = EMB // L

NBUF = 4

_mesh = plsc.VectorSubcoreMesh(
    core_axis_name="c", subcore_axis_name="s", num_cores=NC, num_subcores=NS)


@functools.partial(
    pl.kernel,
    out_type=jax.ShapeDtypeStruct((BATCH, EMB), jnp.float32),
    mesh=_mesh,
    scratch_types=[
        pltpu.VMEM((ROWS_PER_W * 2, CHUNK), jnp.int32),
        pltpu.VMEM((NBUF, 2, CHUNK, EMB), jnp.float32),
        pltpu.VMEM((ROWS_PER_W, EMB), jnp.float32),
        [pltpu.SemaphoreType.DMA] * NBUF,
    ],
    compiler_params=pltpu.CompilerParams(use_tc_tiling_on_sc=False),
)
def _pool_kernel(idx_hbm, table_hbm, out_hbm, idx_v, buf, out_v, sems):
    wid = lax.axis_index("s") * NC + lax.axis_index("c")
    base2 = wid * (ROWS_PER_W * 2)
    pltpu.sync_copy(idx_hbm.at[pl.ds(base2, ROWS_PER_W * 2)], idx_v)

    neg = jnp.full((L,), -jnp.inf, dtype=jnp.float32)

    def fire(row, slot):
        pltpu.async_copy(table_hbm.at[idx_v.at[2 * row]], buf.at[slot, 0],
                         sems[slot])
        pltpu.async_copy(table_hbm.at[idx_v.at[2 * row + 1]], buf.at[slot, 1],
                         sems[slot])

    def drain(slot):
        for c in range(2):
            pltpu.make_async_copy(table_hbm.at[pl.ds(0, CHUNK)],
                                  buf.at[slot, c], sems[slot]).wait()

    def row_max(b0, b1):
        def jbody(j, a):
            a = tuple(
                jnp.maximum(a[d], b0[j, pl.ds(L * d, L)]) for d in range(EV))
            return tuple(
                jnp.maximum(a[d], b1[j, pl.ds(L * d, L)]) for d in range(EV))
        return lax.fori_loop(0, CHUNK, jbody, (neg,) * EV, unroll=4)

    for slot in range(NBUF):
        fire(slot, slot)

    def outer_body(k, carry):
        for p in range(NBUF):
            r = NBUF * k + p
            drain(p)
            acc = row_max(buf.at[p, 0], buf.at[p, 1])
            for d in range(EV):
                out_v[r, pl.ds(L * d, L)] = acc[d]
            fire(lax.rem(r + NBUF, ROWS_PER_W), p)
        return carry

    lax.fori_loop(0, ROWS_PER_W // NBUF, outer_body, 0)
    for slot in range(NBUF):
        drain(slot)
    pltpu.sync_copy(out_v, out_hbm.at[pl.ds(wid * ROWS_PER_W, ROWS_PER_W)])


def _matmul_body(x_ref, wt_ref, b_ref, o_ref):
    o_ref[:] = (
        jnp.dot(x_ref[:], wt_ref[:], preferred_element_type=jnp.float32)
        + b_ref[:])


_matmul = pl.pallas_call(
    _matmul_body,
    out_shape=jax.ShapeDtypeStruct((BATCH, NCLS), jnp.float32),
)


def kernel(content, emb_table, W, b):
    idx = content.reshape(BATCH * 2, CHUNK)
    pooled = _pool_kernel(idx, emb_table)
    return _matmul(pooled, W.T, b.reshape(1, NCLS))


# speedup vs baseline: 1.0754x; 1.0030x over previous
import functools

import jax
import jax.numpy as jnp
from jax import lax
from jax.experimental import pallas as pl
from jax.experimental.pallas import tpu as pltpu
from jax.experimental.pallas import tpu_sc as plsc

BATCH = 4096
SEQ = 200
EMB = 64
NCLS = 8
VOCAB = 1000000

NC = 2
NS = 16
NW = NC * NS
ROWS_PER_W = BATCH // NW
CHUNK = SEQ // 2
L = 16
EV = EMB // L

NBUF = 4

_mesh = plsc.VectorSubcoreMesh(
    core_axis_name="c", subcore_axis_name="s", num_cores=NC, num_subcores=NS)


@functools.partial(
    pl.kernel,
    out_type=jax.ShapeDtypeStruct((BATCH, EMB), jnp.float32),
    mesh=_mesh,
    scratch_types=[
        pltpu.VMEM((ROWS_PER_W * 2, CHUNK), jnp.int32),
        pltpu.VMEM((NBUF, 2, CHUNK, EMB), jnp.float32),
        pltpu.VMEM((ROWS_PER_W, EMB), jnp.float32),
        [pltpu.SemaphoreType.DMA] * NBUF,
    ],
    compiler_params=pltpu.CompilerParams(use_tc_tiling_on_sc=False),
)
def _pool_kernel(idx_hbm, table_hbm, out_hbm, idx_v, buf, out_v, sems):
    wid = lax.axis_index("s") * NC + lax.axis_index("c")
    base2 = wid * (ROWS_PER_W * 2)
    pltpu.sync_copy(idx_hbm.at[pl.ds(base2, ROWS_PER_W * 2)], idx_v)

    neg = jnp.full((L,), -jnp.inf, dtype=jnp.float32)

    def fire(row, slot):
        pltpu.async_copy(table_hbm.at[idx_v.at[2 * row]], buf.at[slot, 0],
                         sems[slot])
        pltpu.async_copy(table_hbm.at[idx_v.at[2 * row + 1]], buf.at[slot, 1],
                         sems[slot])

    def drain(slot):
        for c in range(2):
            pltpu.make_async_copy(table_hbm.at[pl.ds(0, CHUNK)],
                                  buf.at[slot, c], sems[slot]).wait()

    def row_max(b0, b1):
        @plsc.parallel_loop(0, CHUNK, unroll=4, carry=(neg,) * (2 * EV))
        def acc(j, a):
            lo = tuple(
                jnp.maximum(a[d], b0[j, pl.ds(L * d, L)]) for d in range(EV))
            hi = tuple(
                jnp.maximum(a[EV + d], b1[j, pl.ds(L * d, L)])
                for d in range(EV))
            return lo + hi
        return tuple(jnp.maximum(acc[d], acc[EV + d]) for d in range(EV))

    for slot in range(NBUF):
        fire(slot, slot)

    def outer_body(k, carry):
        for p in range(NBUF):
            r = NBUF * k + p
            drain(p)
            acc = row_max(buf.at[p, 0], buf.at[p, 1])
            for d in range(EV):
                out_v[r, pl.ds(L * d, L)] = acc[d]
            fire(lax.rem(r + NBUF, ROWS_PER_W), p)
        return carry

    lax.fori_loop(0, ROWS_PER_W // NBUF, outer_body, 0)
    for slot in range(NBUF):
        drain(slot)
    pltpu.sync_copy(out_v, out_hbm.at[pl.ds(wid * ROWS_PER_W, ROWS_PER_W)])


def _matmul_body(x_ref, wt_ref, b_ref, o_ref):
    o_ref[:] = (
        jnp.dot(x_ref[:], wt_ref[:], preferred_element_type=jnp.float32)
        + b_ref[:])


_matmul = pl.pallas_call(
    _matmul_body,
    out_shape=jax.ShapeDtypeStruct((BATCH, NCLS), jnp.float32),
)


def kernel(content, emb_table, W, b):
    idx = content.reshape(BATCH * 2, CHUNK)
    pooled = _pool_kernel(idx, emb_table)
    return _matmul(pooled, W.T, b.reshape(1, NCLS))
